# baseline (speedup 1.0000x reference)
import functools

import numpy as np
import jax
import jax.numpy as jnp
from jax import lax
from jax.experimental import pallas as pl
from jax.experimental.pallas import tpu as pltpu
from jax.experimental.pallas import tpu_sc as plsc

V = 1000000
D = 64
B = 4096
L = 200
W = 2
K = 2 * W + 1

NC = 2
NS = 16
NW = NC * NS
BPW = B // NW
RPW = BPW * K
NLANES = 16
NGROUP = RPW // NLANES
CHUNK = 128
NCHUNK = RPW // CHUNK


NBLK = 7812
TPB = 245
VTAIL = NBLK * 128


def _sc_transpose(table_t, tail):
  mesh = plsc.VectorSubcoreMesh(core_axis_name="c", subcore_axis_name="s",
                                num_cores=NC, num_subcores=NS)

  @functools.partial(
      pl.kernel,
      mesh=mesh,
      compiler_params=pltpu.CompilerParams(needs_layout_passes=False),
      out_type=jax.ShapeDtypeStruct((V // 2, 2 * D), jnp.float32),
      scratch_types=[
          [pltpu.VMEM((D, 128), jnp.float32) for _ in range(2)],
          [pltpu.VMEM((64, 2 * D), jnp.float32) for _ in range(2)],
          pltpu.VMEM((64, D), jnp.float32),
          [pltpu.SemaphoreType.DMA for _ in range(4)],
      ],
  )
  def k(tab_hbm, tail_hbm, out_hbm, stages, outbs, tailv, sems):
    wid = lax.axis_index("s") * NC + lax.axis_index("c")
    s0, s1, w0, w1 = sems
    lanes = lax.iota(jnp.int32, NLANES)

    def src(blk):
      return tab_hbm.at[:, pl.ds(pl.multiple_of(blk * 128, 128), 128)]

    def process(stage, outb):
      def row(pp, carry):
        for h in range(2):
          col = lanes * 0 + (2 * pp + h)
          for c4 in range(D // NLANES):
            x = plsc.load_gather(stage, [c4 * NLANES + lanes, col])
            outb[pp, pl.ds(h * D + c4 * NLANES, NLANES)] = x
        return carry
      lax.fori_loop(0, 64, row, 0)

    pltpu.async_copy(src(wid), stages[0], s0)

    def body(it, carry):
      blk0 = (2 * it) * NW + wid
      blk1 = (2 * it + 1) * NW + wid
      blk2 = (2 * it + 2) * NW + wid

      @pl.when(blk1 < NBLK)
      def _pf_odd():
        pltpu.async_copy(src(blk1), stages[1], s1)

      @pl.when(blk0 < NBLK)
      def _even():
        pltpu.make_async_copy(src(0), stages[0], s0).wait()

        @pl.when(it > 0)
        def _wb():
          pltpu.make_async_copy(src(0), outbs[0], w0).wait()
        process(stages[0], outbs[0])
        pltpu.async_copy(
            outbs[0],
            out_hbm.at[pl.ds(pl.multiple_of(blk0 * 64, 8), 64), :], w0)

      @pl.when(blk2 < NBLK)
      def _pf_even():
        pltpu.async_copy(src(blk2), stages[0], s0)

      @pl.when(blk1 < NBLK)
      def _odd():
        pltpu.make_async_copy(src(0), stages[1], s1).wait()

        @pl.when(it > 0)
        def _wb2():
          pltpu.make_async_copy(src(0), outbs[1], w1).wait()
        process(stages[1], outbs[1])
        pltpu.async_copy(
            outbs[1],
            out_hbm.at[pl.ds(pl.multiple_of(blk1 * 64, 8), 64), :], w1)
      return carry

    lax.fori_loop(0, (TPB + 1) // 2, body, 0)
    pltpu.make_async_copy(src(0), outbs[0], w0).wait()
    pltpu.make_async_copy(src(0), outbs[1], w1).wait()

    @pl.when(wid == 0)
    def _tail():
      pltpu.sync_copy(tail_hbm, tailv)
      for pp in range(32):
        for h in range(2):
          for c4 in range(D // NLANES):
            x = plsc.load_gather(
                tailv, [lanes * 0 + (2 * pp + h), c4 * NLANES + lanes])
            outbs[0][pp, pl.ds(h * D + c4 * NLANES, NLANES)] = x
      pltpu.sync_copy(outbs[0].at[pl.ds(0, 32), :],
                      out_hbm.at[pl.ds(VTAIL // 2, 32), :])

  return k(table_t, tail)


def _sc_window_gather(table_pairs, idx_flat, anchor_index):
  mesh = plsc.VectorSubcoreMesh(core_axis_name="c", subcore_axis_name="s",
                                num_cores=NC, num_subcores=NS)

  r_all = np.arange(RPW, dtype=np.int32)
  b_map = jnp.asarray(r_all // K)
  off_map = jnp.asarray(r_all % K - W)

  @functools.partial(
      pl.kernel,
      mesh=mesh,
      compiler_params=pltpu.CompilerParams(needs_layout_passes=False),
      out_type=jax.ShapeDtypeStruct((B * K, D), jnp.float32),
      scratch_types=[
          pltpu.VMEM((BPW * L + BPW,), jnp.int32),
          pltpu.VMEM((RPW,), jnp.int32),
          pltpu.VMEM((RPW,), jnp.int32),
          [pltpu.VMEM((CHUNK,), jnp.int32) for _ in range(NCHUNK)],
          [pltpu.VMEM((CHUNK,), jnp.int32) for _ in range(NCHUNK)],
          [pltpu.VMEM((CHUNK, 2 * D), jnp.float32) for _ in range(NCHUNK)],
          pltpu.VMEM((CHUNK, D), jnp.float32),
          pltpu.SemaphoreType.DMA,
      ],
  )
  def k(tab_hbm, idx_hbm, anc_hbm, bmap_hbm, omap_hbm, out_hbm,
        idx_v, bmap_v, omap_v, tids, pks, rows, outv, sem):
    wid = lax.axis_index("s") * NC + lax.axis_index("c")
    base = wid * BPW
    anc_off = BPW * L
    pltpu.sync_copy(idx_hbm.at[pl.ds(base * L, BPW * L)],
                    idx_v.at[pl.ds(0, BPW * L)])
    pltpu.sync_copy(anc_hbm.at[pl.ds(base, BPW)],
                    idx_v.at[pl.ds(anc_off, BPW)])
    pltpu.sync_copy(bmap_hbm, bmap_v)
    pltpu.sync_copy(omap_hbm, omap_v)

    lanes = lax.iota(jnp.int32, NLANES)
    gpc = CHUNK // NLANES
    for g in range(NGROUP):
      b_loc = bmap_v[pl.ds(g * NLANES, NLANES)]
      off = omap_v[pl.ds(g * NLANES, NLANES)]
      a = plsc.load_gather(idx_v, [anc_off + b_loc])
      pos = a + off
      valid = ((pos >= 0) & (pos < L)).astype(jnp.int32)
      posc = jnp.clip(pos, 0, L - 1)
      tok = plsc.load_gather(idx_v, [b_loc * L + posc])
      sl = pl.ds((g % gpc) * NLANES, NLANES)
      tids[g // gpc][sl] = tok >> 1
      pks[g // gpc][sl] = (tok & 1) + (valid << 1)

    copies = [pltpu.async_copy(tab_hbm.at[tids[c]], rows[c], sem)
              for c in range(NCHUNK)]

    for c in range(NCHUNK):
      copies[c].wait()

      def extract(i, carry, c=c):
        pkv = pks[c][pl.ds((i // NLANES) * NLANES, NLANES)]
        pk = jnp.sum(jnp.where(lanes == (i % NLANES), pkv, 0))
        hf = (pk & 1).astype(jnp.float32)
        vf = ((pk >> 1) & 1).astype(jnp.float32)
        wlo = (1.0 - hf) * vf
        whi = hf * vf
        for c4 in range(D // NLANES):
          xlo = rows[c][i, pl.ds(c4 * NLANES, NLANES)]
          xhi = rows[c][i, pl.ds(D + c4 * NLANES, NLANES)]
          outv[i, pl.ds(c4 * NLANES, NLANES)] = xlo * wlo + xhi * whi
        return carry
      lax.fori_loop(0, CHUNK, extract, 0)
      pltpu.sync_copy(outv, out_hbm.at[pl.ds(base * K + c * CHUNK, CHUNK), :])

  return k(table_pairs, idx_flat, anchor_index, b_map, off_map)


def kernel(table, indices, anchor_index):
  tab_pairs = _sc_transpose(table.T, lax.slice(table, (VTAIL, 0), (V, D)))
  out = _sc_window_gather(tab_pairs, indices.reshape(-1), anchor_index)
  return out.reshape(B, K * D)


# speedup vs baseline: 2.4419x; 2.4419x over previous
import functools

import numpy as np
import jax
import jax.numpy as jnp
from jax import lax
from jax.experimental import pallas as pl
from jax.experimental.pallas import tpu as pltpu
from jax.experimental.pallas import tpu_sc as plsc

D = 64
B = 4096
L = 200
W = 2
K = 2 * W + 1

NC = 2
NS = 16
NW = NC * NS
BPW = B // NW
NLANES = 16
NGROUP = (BPW * K) // NLANES


def _sc_window_gather(table, idx_flat, anchor_index):
  mesh = plsc.VectorSubcoreMesh(core_axis_name="c", subcore_axis_name="s",
                                num_cores=NC, num_subcores=NS)

  r_all = np.arange(NGROUP * NLANES, dtype=np.int32)
  b_map = jnp.asarray(r_all // K)
  off_map = jnp.asarray(r_all % K - W)

  @functools.partial(
      pl.kernel,
      mesh=mesh,
      compiler_params=pltpu.CompilerParams(needs_layout_passes=False,
                                           use_tc_tiling_on_sc=False),
      out_type=jax.ShapeDtypeStruct((B * K, D), jnp.float32),
      scratch_types=[
          pltpu.VMEM((BPW * L + BPW,), jnp.int32),
          pltpu.VMEM((K * BPW,), jnp.int32),
          pltpu.VMEM((K * BPW,), jnp.int32),
          [pltpu.VMEM((BPW,), jnp.int32) for _ in range(K)],
          pltpu.VMEM((K * BPW, D), jnp.float32),
          pltpu.SemaphoreType.DMA,
      ],
  )
  def k(table_hbm, idx_hbm, anc_hbm, bmap_hbm, omap_hbm, out_hbm,
        idx_v, bmap_v, omap_v, toks, rows_v, sem):
    wid = lax.axis_index("s") * NC + lax.axis_index("c")
    base = wid * BPW
    anc_off = BPW * L
    pltpu.sync_copy(idx_hbm.at[pl.ds(base * L, BPW * L)],
                    idx_v.at[pl.ds(0, BPW * L)])
    pltpu.sync_copy(anc_hbm.at[pl.ds(base, BPW)],
                    idx_v.at[pl.ds(anc_off, BPW)])
    pltpu.sync_copy(bmap_hbm, bmap_v)
    pltpu.sync_copy(omap_hbm, omap_v)

    lanes = lax.iota(jnp.int32, NLANES)
    for g in range(NGROUP):
      b_loc = bmap_v[pl.ds(g * NLANES, NLANES)]
      off = omap_v[pl.ds(g * NLANES, NLANES)]
      a = plsc.load_gather(idx_v, [anc_off + b_loc])
      pos = jnp.clip(a + off, 0, L - 1)
      tok = plsc.load_gather(idx_v, [b_loc * L + pos])
      toks[g * NLANES // BPW][pl.ds((g * NLANES) % BPW, NLANES)] = tok

    copies = [
        pltpu.async_copy(table_hbm.at[toks[kk]],
                         rows_v.at[pl.ds(kk * BPW, BPW), :], sem)
        for kk in range(K)
    ]
    for c in copies:
      c.wait()

    zero16 = jnp.zeros((NLANES,), jnp.float32)
    for g in range(NGROUP):
      b_loc = bmap_v[pl.ds(g * NLANES, NLANES)]
      off = omap_v[pl.ds(g * NLANES, NLANES)]
      a = plsc.load_gather(idx_v, [anc_off + b_loc])
      pos = a + off
      inv = ((pos < 0) | (pos >= L)).astype(jnp.int32)
      n_inv = jnp.sum(inv)

      @pl.when(n_inv > 0)
      def _fix(g=g, inv=inv):
        def body(lane, carry):
          bad = jnp.sum(jnp.where(lanes == lane, inv, 0)) > 0

          @pl.when(bad)
          def _zero():
            rr = g * NLANES + lane
            for c4 in range(D // NLANES):
              rows_v[rr, pl.ds(c4 * NLANES, NLANES)] = zero16
          return carry
        lax.fori_loop(0, NLANES, body, 0)

    pltpu.sync_copy(rows_v, out_hbm.at[pl.ds(base * K, BPW * K), :])

  return k(table, idx_flat, anchor_index, b_map, off_map)


def kernel(table, indices, anchor_index):
  out = _sc_window_gather(table, indices.reshape(-1), anchor_index)
  return out.reshape(B, K * D)
